# baseline (speedup 1.0000x reference)
import functools

import jax
import jax.numpy as jnp
from jax import lax
from jax.experimental import pallas as pl
from jax.experimental.pallas import tpu as pltpu
from jax.experimental.pallas import tpu_sc as plsc

D = 32
L = 16


@functools.cache
def _make_gather(BATCH: int, HIST: int):
    info = plsc.get_sparse_core_info()
    NC, NS = info.num_cores, info.num_subcores
    NW = NC * NS
    BPW = BATCH // NW
    NG = BPW // 128

    mesh = plsc.VectorSubcoreMesh(core_axis_name="c", subcore_axis_name="s")

    @functools.partial(
        pl.kernel,
        mesh=mesh,
        compiler_params=pltpu.CompilerParams(
            use_tc_tiling_on_sc=False, needs_layout_passes=False
        ),
        out_type=jax.ShapeDtypeStruct((HIST, D, BATCH), jnp.float32),
        scratch_types=[
            pltpu.VMEM((HIST, BPW), jnp.int32),
            pltpu.VMEM((2, BPW, D), jnp.float32),
            pltpu.VMEM((2, D, BPW), jnp.float32),
            pltpu.SemaphoreType.DMA,
            pltpu.SemaphoreType.DMA,
            pltpu.SemaphoreType.DMA,
        ],
    )
    def k(table_hbm, idx_hbm, out_hbm, idx_v, gbuf, tbuf, gsem0, gsem1, wsem):
        wid = lax.axis_index("s") * NC + lax.axis_index("c")
        b0 = wid * BPW
        pltpu.sync_copy(idx_hbm.at[:, pl.ds(b0, BPW)], idx_v)

        gsems = (gsem0, gsem1)
        bvecs = [j * L + lax.iota(jnp.int32, 16) for j in range(BPW // L)]

        def fire(h, gb):
            for j in range(NG):
                pltpu.async_copy(
                    table_hbm.at[idx_v.at[h, pl.ds(j * 128, 128)]],
                    gbuf.at[gb, pl.ds(j * 128, 128)],
                    gsems[gb],
                )

        def drain_gather(gb):
            for j in range(NG):
                pltpu.make_async_copy(
                    table_hbm.at[pl.ds(0, 128)],
                    gbuf.at[gb, pl.ds(j * 128, 128)],
                    gsems[gb],
                ).wait()

        def drain_write(gb):
            pltpu.make_async_copy(
                out_hbm.at[0, :, pl.ds(b0, BPW)], tbuf.at[gb], wsem
            ).wait()

        def step(h, gb, p):
            @pl.when(h + 1 < HIST)
            def _():
                fire(h + 1, 1 - gb)

            drain_gather(gb)

            @pl.when(p >= 1)
            def _():
                drain_write(gb)

            src = gbuf.at[gb]
            dst = tbuf.at[gb]

            def tr_col(e, c):
                ev = jnp.full((16,), 0, jnp.int32) + e
                for j in range(BPW // L):
                    vals = plsc.load_gather(src, [bvecs[j], ev])
                    dst[e, pl.ds(j * L, L)] = vals
                return c

            lax.fori_loop(0, D, tr_col, 0)
            pltpu.async_copy(dst, out_hbm.at[h, :, pl.ds(b0, BPW)], wsem)

        fire(0, 0)

        def pair(p, carry):
            step(2 * p, 0, p)
            step(2 * p + 1, 1, p)
            return carry

        lax.fori_loop(0, HIST // 2, pair, 0)
        drain_write(0)
        drain_write(1)

    return k


def kernel(inputs, table):
    BATCH, HIST = inputs.shape
    idx_t = inputs.astype(jnp.int32).T
    out = _make_gather(BATCH, HIST)(table, idx_t)
    return out.transpose(2, 0, 1)


# speedup vs baseline: 1.1581x; 1.1581x over previous
import functools

import jax
import jax.numpy as jnp
from jax import lax
from jax.experimental import pallas as pl
from jax.experimental.pallas import tpu as pltpu
from jax.experimental.pallas import tpu_sc as plsc

D = 32
L = 16


@functools.cache
def _make_gather(BATCH: int, HIST: int):
    info = plsc.get_sparse_core_info()
    NC, NS = info.num_cores, info.num_subcores
    NW = NC * NS
    BPW = BATCH // NW
    NG = BPW // 128

    mesh = plsc.VectorSubcoreMesh(core_axis_name="c", subcore_axis_name="s")

    @functools.partial(
        pl.kernel,
        mesh=mesh,
        compiler_params=pltpu.CompilerParams(
            use_tc_tiling_on_sc=False, needs_layout_passes=False
        ),
        out_type=jax.ShapeDtypeStruct((HIST, D, BATCH), jnp.float32),
        scratch_types=[
            pltpu.VMEM((HIST, BPW), jnp.int32),
            pltpu.VMEM((2, BPW, D), jnp.float32),
            pltpu.VMEM((BPW, D + 1), jnp.float32),
            pltpu.VMEM((2, D, BPW), jnp.float32),
            pltpu.SemaphoreType.DMA,
            pltpu.SemaphoreType.DMA,
            pltpu.SemaphoreType.DMA,
        ],
    )
    def k(table_hbm, idx_hbm, out_hbm, idx_v, gbuf, pbuf, tbuf, gsem0, gsem1, wsem):
        wid = lax.axis_index("s") * NC + lax.axis_index("c")
        b0 = wid * BPW
        pltpu.sync_copy(idx_hbm.at[:, pl.ds(b0, BPW)], idx_v)

        gsems = (gsem0, gsem1)
        bvecs = [j * L + lax.iota(jnp.int32, 16) for j in range(BPW // L)]

        def fire(h, gb):
            for j in range(NG):
                pltpu.async_copy(
                    table_hbm.at[idx_v.at[h, pl.ds(j * 128, 128)]],
                    gbuf.at[gb, pl.ds(j * 128, 128)],
                    gsems[gb],
                )

        def drain_gather(gb):
            for j in range(NG):
                pltpu.make_async_copy(
                    table_hbm.at[pl.ds(0, 128)],
                    gbuf.at[gb, pl.ds(j * 128, 128)],
                    gsems[gb],
                ).wait()

        def drain_write(gb):
            pltpu.make_async_copy(
                out_hbm.at[0, :, pl.ds(b0, BPW)], tbuf.at[gb], wsem
            ).wait()

        def step(h, gb, p):
            @pl.when(h + 1 < HIST)
            def _():
                fire(h + 1, 1 - gb)

            drain_gather(gb)

            @pl.when(p >= 1)
            def _():
                drain_write(gb)

            src = gbuf.at[gb]
            dst = tbuf.at[gb]

            def pad_row(bb, c):
                for j in range(4):
                    b = bb * 4 + j
                    pbuf[b, pl.ds(0, L)] = src[b, pl.ds(0, L)]
                    pbuf[b, pl.ds(L, L)] = src[b, pl.ds(L, L)]
                return c

            lax.fori_loop(0, BPW // 4, pad_row, 0)

            def tr_col(e, c):
                ev = jnp.full((16,), 0, jnp.int32) + e
                for j in range(BPW // L):
                    vals = plsc.load_gather(pbuf, [bvecs[j], ev])
                    dst[e, pl.ds(j * L, L)] = vals
                return c

            lax.fori_loop(0, D, tr_col, 0)
            pltpu.async_copy(dst, out_hbm.at[h, :, pl.ds(b0, BPW)], wsem)

        fire(0, 0)

        def pair(p, carry):
            step(2 * p, 0, p)
            step(2 * p + 1, 1, p)
            return carry

        lax.fori_loop(0, HIST // 2, pair, 0)
        drain_write(0)
        drain_write(1)

    return k


def kernel(inputs, table):
    BATCH, HIST = inputs.shape
    idx_t = inputs.astype(jnp.int32).T
    out = _make_gather(BATCH, HIST)(table, idx_t)
    return out.transpose(2, 0, 1)


# speedup vs baseline: 1.5306x; 1.3216x over previous
import functools

import jax
import jax.numpy as jnp
from jax import lax
from jax.experimental import pallas as pl
from jax.experimental.pallas import tpu as pltpu
from jax.experimental.pallas import tpu_sc as plsc

D = 32
L = 16


@functools.cache
def _make_gather(BATCH: int, HIST: int):
    info = plsc.get_sparse_core_info()
    NC, NS = info.num_cores, info.num_subcores
    NW = NC * NS
    BPW = BATCH // NW
    NG = BPW // 128

    mesh = plsc.VectorSubcoreMesh(core_axis_name="c", subcore_axis_name="s")

    @functools.partial(
        pl.kernel,
        mesh=mesh,
        compiler_params=pltpu.CompilerParams(
            use_tc_tiling_on_sc=False, needs_layout_passes=False
        ),
        out_type=jax.ShapeDtypeStruct((HIST, D, BATCH), jnp.float32),
        scratch_types=[
            pltpu.VMEM((HIST, BPW), jnp.int32),
            pltpu.VMEM((2, BPW, D), jnp.float32),
            pltpu.VMEM((2, D, BPW + 1), jnp.float32),
            pltpu.SemaphoreType.DMA,
            pltpu.SemaphoreType.DMA,
            pltpu.SemaphoreType.DMA,
        ],
    )
    def k(table_hbm, idx_hbm, out_hbm, idx_v, gbuf, tbuf, gsem0, gsem1, wsem):
        wid = lax.axis_index("s") * NC + lax.axis_index("c")
        b0 = wid * BPW
        pltpu.sync_copy(idx_hbm.at[:, pl.ds(b0, BPW)], idx_v)

        gsems = (gsem0, gsem1)
        evecs = [eh * L + lax.iota(jnp.int32, 16) for eh in range(D // L)]

        def fire(h, gb):
            for j in range(NG):
                pltpu.async_copy(
                    table_hbm.at[idx_v.at[h, pl.ds(j * 128, 128)]],
                    gbuf.at[gb, pl.ds(j * 128, 128)],
                    gsems[gb],
                )

        def drain_gather(gb):
            for j in range(NG):
                pltpu.make_async_copy(
                    table_hbm.at[pl.ds(0, 128)],
                    gbuf.at[gb, pl.ds(j * 128, 128)],
                    gsems[gb],
                ).wait()

        def drain_write(gb):
            pltpu.make_async_copy(
                out_hbm.at[0, :, pl.ds(b0, BPW)],
                tbuf.at[gb, :, pl.ds(0, BPW)],
                wsem,
            ).wait()

        def step(h, gb, p):
            @pl.when(h + 1 < HIST)
            def _():
                fire(h + 1, 1 - gb)

            drain_gather(gb)

            @pl.when(p >= 1)
            def _():
                drain_write(gb)

            src = gbuf.at[gb]
            dst = tbuf.at[gb]

            def tr_rows(bb, c):
                for u in range(8):
                    b = bb * 8 + u
                    bv = jnp.full((16,), 0, jnp.int32) + b
                    for eh in range(D // L):
                        vals = src[b, pl.ds(eh * L, L)]
                        plsc.store_scatter(dst, [evecs[eh], bv], vals)
                return c

            lax.fori_loop(0, BPW // 8, tr_rows, 0)
            pltpu.async_copy(
                tbuf.at[gb, :, pl.ds(0, BPW)],
                out_hbm.at[h, :, pl.ds(b0, BPW)],
                wsem,
            )

        fire(0, 0)

        def pair(p, carry):
            step(2 * p, 0, p)
            step(2 * p + 1, 1, p)
            return carry

        lax.fori_loop(0, HIST // 2, pair, 0)
        drain_write(0)
        drain_write(1)

    return k


def kernel(inputs, table):
    BATCH, HIST = inputs.shape
    idx_t = inputs.astype(jnp.int32).T
    out = _make_gather(BATCH, HIST)(table, idx_t)
    return out.transpose(2, 0, 1)


# speedup vs baseline: 1.7843x; 1.1657x over previous
import functools

import jax
import jax.numpy as jnp
from jax import lax
from jax.experimental import pallas as pl
from jax.experimental.pallas import tpu as pltpu
from jax.experimental.pallas import tpu_sc as plsc

D = 32
L = 16


@functools.cache
def _make_gather(BATCH: int, HIST: int):
    info = plsc.get_sparse_core_info()
    NC, NS = info.num_cores, info.num_subcores
    NW = NC * NS
    BPW = BATCH // NW
    NG = BPW // 128

    mesh = plsc.VectorSubcoreMesh(core_axis_name="c", subcore_axis_name="s")

    @functools.partial(
        pl.kernel,
        mesh=mesh,
        compiler_params=pltpu.CompilerParams(
            use_tc_tiling_on_sc=False, needs_layout_passes=False
        ),
        out_type=jax.ShapeDtypeStruct((HIST, D, BATCH), jnp.float32),
        scratch_types=[
            pltpu.VMEM((HIST, BPW), jnp.int32),
            pltpu.VMEM((2, BPW, D), jnp.float32),
            pltpu.VMEM((2, D, BPW + 1), jnp.float32),
            pltpu.SemaphoreType.DMA,
            pltpu.SemaphoreType.DMA,
            pltpu.SemaphoreType.DMA,
        ],
    )
    def k(table_hbm, idx_hbm, out_hbm, idx_v, gbuf, tbuf, gsem0, gsem1, wsem):
        wid = lax.axis_index("s") * NC + lax.axis_index("c")
        b0 = wid * BPW
        pltpu.sync_copy(idx_hbm.at[:, pl.ds(b0, BPW)], idx_v)

        gsems = (gsem0, gsem1)
        evecs = [eh * L + lax.iota(jnp.int32, 16) for eh in range(D // L)]

        def fire(h, gb):
            for j in range(NG):
                pltpu.async_copy(
                    table_hbm.at[idx_v.at[h, pl.ds(j * 128, 128)]],
                    gbuf.at[gb, pl.ds(j * 128, 128)],
                    gsems[gb],
                )

        def drain_gather(gb):
            for j in range(NG):
                pltpu.make_async_copy(
                    table_hbm.at[pl.ds(0, 128)],
                    gbuf.at[gb, pl.ds(j * 128, 128)],
                    gsems[gb],
                ).wait()

        def drain_write(gb):
            pltpu.make_async_copy(
                out_hbm.at[0, :, pl.ds(b0, BPW)],
                tbuf.at[gb, :, pl.ds(0, BPW)],
                wsem,
            ).wait()

        def step(h, gb, p):
            @pl.when(h + 1 < HIST)
            def _():
                fire(h + 1, 1 - gb)

            drain_gather(gb)

            @pl.when(p >= 1)
            def _():
                drain_write(gb)

            src = gbuf.at[gb]
            dst = tbuf.at[gb]

            @plsc.parallel_loop(0, BPW, 1, unroll=8)
            def tr_rows(b):
                bv = jnp.full((16,), 0, jnp.int32) + b
                for eh in range(D // L):
                    vals = src[b, pl.ds(eh * L, L)]
                    plsc.store_scatter(dst, [evecs[eh], bv], vals)
            pltpu.async_copy(
                tbuf.at[gb, :, pl.ds(0, BPW)],
                out_hbm.at[h, :, pl.ds(b0, BPW)],
                wsem,
            )

        fire(0, 0)

        def pair(p, carry):
            step(2 * p, 0, p)
            step(2 * p + 1, 1, p)
            return carry

        lax.fori_loop(0, HIST // 2, pair, 0)
        drain_write(0)
        drain_write(1)

    return k


def kernel(inputs, table):
    BATCH, HIST = inputs.shape
    idx_t = inputs.astype(jnp.int32).T
    out = _make_gather(BATCH, HIST)(table, idx_t)
    return out.transpose(2, 0, 1)
